# baseline (speedup 1.0000x reference)
import functools

import jax
import jax.numpy as jnp
from jax import lax
from jax.experimental import pallas as pl
from jax.experimental.pallas import tpu as pltpu
from jax.experimental.pallas import tpu_sc as plsc

NC = 2
NS = 16
NW = NC * NS
B = 24576
BPW = B // NW


def _make_gather_grouped(Dp: int, R: int):
    nchunks = BPW // R
    ngroups = R // 6
    mesh = plsc.VectorSubcoreMesh(core_axis_name="c", subcore_axis_name="s")

    @functools.partial(
        pl.kernel,
        out_type=jax.ShapeDtypeStruct((4096 * 8, Dp), jnp.float32),
        mesh=mesh,
        scratch_types=[
            pltpu.VMEM((2, R), jnp.int32),
            pltpu.VMEM((2, R, Dp), jnp.float32),
            pltpu.SemaphoreType.DMA,
        ],
        compiler_params=pltpu.CompilerParams(use_tc_tiling_on_sc=False),
    )
    def k(idx_hbm, table_hbm, out_hbm, idx2, rows2, sem):
        wid = lax.axis_index("s") * NC + lax.axis_index("c")
        base = wid * BPW

        pltpu.sync_copy(idx_hbm.at[pl.ds(base, R)], idx2.at[0])
        pltpu.async_copy(table_hbm.at[idx2.at[0]], rows2.at[0], sem)

        def body(c, carry):
            slot = lax.rem(c, 2)
            nslot = lax.rem(c + 1, 2)

            @pl.when(c + 1 < nchunks)
            def _():
                off_n = base + pl.multiple_of((c + 1) * R, 24)
                pltpu.sync_copy(idx_hbm.at[pl.ds(off_n, R)], idx2.at[nslot])
                pltpu.async_copy(
                    table_hbm.at[idx2.at[nslot]], rows2.at[nslot], sem
                )

            pltpu.make_async_copy(
                table_hbm.at[idx2.at[slot]], rows2.at[slot], sem
            ).wait()
            off = base + pl.multiple_of(c * R, 24)
            g0 = off // 6
            for k_ in range(ngroups):
                pltpu.sync_copy(
                    rows2.at[slot, pl.ds(6 * k_, 6)],
                    out_hbm.at[pl.ds(8 * (g0 + k_), 6)],
                )
            return carry

        lax.fori_loop(0, nchunks, body, 0)

    return k


def _make_gather_staged(Dp: int, R: int, V: int = 2048):
    nchunks = BPW // R
    ngroups = R // 6
    rows_per_tile = V // NS
    mesh = plsc.VectorSubcoreMesh(core_axis_name="c", subcore_axis_name="s")

    @functools.partial(
        pl.kernel,
        out_type=jax.ShapeDtypeStruct((4096 * 8, Dp), jnp.float32),
        mesh=mesh,
        scratch_types=[
            pltpu.VMEM((R,), jnp.int32),
            pltpu.VMEM((R, Dp), jnp.float32),
            pltpu.SemaphoreType.DMA,
            pltpu.VMEM_SHARED((V, Dp), jnp.float32),
        ],
        compiler_params=pltpu.CompilerParams(use_tc_tiling_on_sc=False),
    )
    def k(idx_hbm, table_hbm, out_hbm, idx_c, rows_v, sem, stage):
        sid = lax.axis_index("s")
        wid = sid * NC + lax.axis_index("c")
        base = wid * BPW
        srow = sid * rows_per_tile
        pltpu.sync_copy(
            table_hbm.at[pl.ds(srow, rows_per_tile)],
            stage.at[pl.ds(srow, rows_per_tile)],
        )
        plsc.subcore_barrier()

        def body(c, carry):
            off = base + pl.multiple_of(c * R, 24)
            pltpu.sync_copy(idx_hbm.at[pl.ds(off, R)], idx_c)
            pltpu.async_copy(stage.at[idx_c], rows_v, sem).wait()
            g0 = off // 6
            for k_ in range(ngroups):
                pltpu.sync_copy(
                    rows_v.at[pl.ds(6 * k_, 6)],
                    out_hbm.at[pl.ds(8 * (g0 + k_), 6)],
                )
            return carry

        lax.fori_loop(0, nchunks, body, 0)

    return k


def _make_gather_flat(Dp: int, R: int):
    nchunks = BPW // R
    mesh = plsc.VectorSubcoreMesh(core_axis_name="c", subcore_axis_name="s")

    @functools.partial(
        pl.kernel,
        out_type=jax.ShapeDtypeStruct((B, Dp), jnp.float32),
        mesh=mesh,
        scratch_types=[
            pltpu.VMEM((R,), jnp.int32),
            pltpu.VMEM((R, Dp), jnp.float32),
            pltpu.SemaphoreType.DMA,
        ],
        compiler_params=pltpu.CompilerParams(use_tc_tiling_on_sc=False),
    )
    def k(idx_hbm, table_hbm, out_hbm, idx_c, rows_v, sem):
        wid = lax.axis_index("s") * NC + lax.axis_index("c")
        base = wid * BPW

        def body(c, carry):
            off = base + pl.multiple_of(c * R, 8)
            pltpu.sync_copy(idx_hbm.at[pl.ds(off, R)], idx_c)
            pltpu.async_copy(table_hbm.at[idx_c], rows_v, sem).wait()
            pltpu.sync_copy(rows_v, out_hbm.at[pl.ds(off, R)])
            return carry

        lax.fori_loop(0, nchunks, body, 0)

    return k


def _pad8(d: int) -> int:
    return (d + 7) // 8 * 8


_KERNELS = {
    2047: _make_gather_grouped(2048, 24),
    1023: _make_gather_grouped(1024, 48),
    511: _make_gather_staged(512, 96),
}
_TERA = _make_gather_flat(_pad8(19), 128)


def kernel(x, species, abilities, items, movesets, teratypes):
    idx = x.reshape(-1).astype(jnp.int32)
    outs = []
    for table in (species, abilities, items, movesets):
        D = table.shape[1]
        Dp = _pad8(D)
        tp = table if Dp == D else jnp.pad(table, ((0, 0), (0, Dp - D)))
        out = _KERNELS[D](idx, tp).reshape(4096, 8, Dp)
        outs.append(lax.slice(out, (0, 0, 0), (4096, 6, D)))
    tp = jnp.pad(teratypes, ((0, 0), (0, _pad8(19) - 19)))
    out = _TERA(idx, tp)
    outs.append(out[:, :19].reshape(x.shape[0], x.shape[1], 19))
    return (outs[0], outs[1], outs[2], outs[3], outs[4])


# speedup vs baseline: 1.0043x; 1.0043x over previous
import functools

import jax
import jax.numpy as jnp
from jax import lax
from jax.experimental import pallas as pl
from jax.experimental.pallas import tpu as pltpu
from jax.experimental.pallas import tpu_sc as plsc

NC = 2
NS = 16
NW = NC * NS
B = 24576
BPW = B // NW


def _make_gather_grouped(Dp: int, R: int):
    nchunks = BPW // R
    ngroups = R // 6
    mesh = plsc.VectorSubcoreMesh(core_axis_name="c", subcore_axis_name="s")

    @functools.partial(
        pl.kernel,
        out_type=jax.ShapeDtypeStruct((4096 * 8, Dp), jnp.float32),
        mesh=mesh,
        scratch_types=[
            pltpu.VMEM((R,), jnp.int32),
            pltpu.VMEM((R, Dp), jnp.float32),
            pltpu.SemaphoreType.DMA,
        ],
        compiler_params=pltpu.CompilerParams(use_tc_tiling_on_sc=False),
    )
    def k(idx_hbm, table_hbm, out_hbm, idx_c, rows_v, sem):
        wid = lax.axis_index("s") * NC + lax.axis_index("c")
        base = wid * BPW

        def body(c, carry):
            off = base + pl.multiple_of(c * R, 24)
            pltpu.sync_copy(idx_hbm.at[pl.ds(off, R)], idx_c)
            pltpu.async_copy(table_hbm.at[idx_c], rows_v, sem).wait()
            g0 = off // 6
            for k_ in range(ngroups):
                pltpu.sync_copy(
                    rows_v.at[pl.ds(6 * k_, 6)],
                    out_hbm.at[pl.ds(8 * (g0 + k_), 6)],
                )
            return carry

        lax.fori_loop(0, nchunks, body, 0)

    return k


def _make_gather_staged(Dp: int, R: int, V: int = 2048):
    nchunks = BPW // R
    ngroups = R // 6
    rows_per_tile = V // NS
    mesh = plsc.VectorSubcoreMesh(core_axis_name="c", subcore_axis_name="s")

    @functools.partial(
        pl.kernel,
        out_type=jax.ShapeDtypeStruct((4096 * 8, Dp), jnp.float32),
        mesh=mesh,
        scratch_types=[
            pltpu.VMEM((R,), jnp.int32),
            pltpu.VMEM((R, Dp), jnp.float32),
            pltpu.SemaphoreType.DMA,
            pltpu.VMEM_SHARED((V, Dp), jnp.float32),
        ],
        compiler_params=pltpu.CompilerParams(use_tc_tiling_on_sc=False),
    )
    def k(idx_hbm, table_hbm, out_hbm, idx_c, rows_v, sem, stage):
        sid = lax.axis_index("s")
        wid = sid * NC + lax.axis_index("c")
        base = wid * BPW
        srow = sid * rows_per_tile
        pltpu.sync_copy(
            table_hbm.at[pl.ds(srow, rows_per_tile)],
            stage.at[pl.ds(srow, rows_per_tile)],
        )
        plsc.subcore_barrier()

        def body(c, carry):
            off = base + pl.multiple_of(c * R, 24)
            pltpu.sync_copy(idx_hbm.at[pl.ds(off, R)], idx_c)
            pltpu.async_copy(stage.at[idx_c], rows_v, sem).wait()
            g0 = off // 6
            for k_ in range(ngroups):
                pltpu.sync_copy(
                    rows_v.at[pl.ds(6 * k_, 6)],
                    out_hbm.at[pl.ds(8 * (g0 + k_), 6)],
                )
            return carry

        lax.fori_loop(0, nchunks, body, 0)

    return k


def _make_gather_flat(Dp: int, R: int):
    nchunks = BPW // R
    mesh = plsc.VectorSubcoreMesh(core_axis_name="c", subcore_axis_name="s")

    @functools.partial(
        pl.kernel,
        out_type=jax.ShapeDtypeStruct((B, Dp), jnp.float32),
        mesh=mesh,
        scratch_types=[
            pltpu.VMEM((R,), jnp.int32),
            pltpu.VMEM((R, Dp), jnp.float32),
            pltpu.SemaphoreType.DMA,
        ],
        compiler_params=pltpu.CompilerParams(use_tc_tiling_on_sc=False),
    )
    def k(idx_hbm, table_hbm, out_hbm, idx_c, rows_v, sem):
        wid = lax.axis_index("s") * NC + lax.axis_index("c")
        base = wid * BPW

        def body(c, carry):
            off = base + pl.multiple_of(c * R, 8)
            pltpu.sync_copy(idx_hbm.at[pl.ds(off, R)], idx_c)
            pltpu.async_copy(table_hbm.at[idx_c], rows_v, sem).wait()
            pltpu.sync_copy(rows_v, out_hbm.at[pl.ds(off, R)])
            return carry

        lax.fori_loop(0, nchunks, body, 0)

    return k


def _pad8(d: int) -> int:
    return (d + 7) // 8 * 8


_KERNELS = {
    2047: _make_gather_grouped(2048, 48),
    1023: _make_gather_grouped(1024, 96),
    511: _make_gather_staged(512, 96),
}
_TERA = _make_gather_flat(_pad8(19), 128)


def kernel(x, species, abilities, items, movesets, teratypes):
    idx = x.reshape(-1).astype(jnp.int32)
    outs = []
    for table in (species, abilities, items, movesets):
        D = table.shape[1]
        Dp = _pad8(D)
        tp = table if Dp == D else jnp.pad(table, ((0, 0), (0, Dp - D)))
        out = _KERNELS[D](idx, tp).reshape(4096, 8, Dp)
        outs.append(lax.slice(out, (0, 0, 0), (4096, 6, D)))
    tp = jnp.pad(teratypes, ((0, 0), (0, _pad8(19) - 19)))
    out = _TERA(idx, tp)
    outs.append(out[:, :19].reshape(x.shape[0], x.shape[1], 19))
    return (outs[0], outs[1], outs[2], outs[3], outs[4])
